# baseline (speedup 1.0000x reference)
import jax
import jax.numpy as jnp
from jax.experimental import pallas as pl
from jax.experimental.pallas import tpu as pltpu

C = 256
QK = 256
VD = 256
S2 = 49
F = C * S2
SOFTMAX_SCALE = 1.0 / (0.01 * C ** 0.5)
EPS = 1e-12
INV_S2 = 1.0 / S2


def _pool(x_f32, p_bf16):
    return jax.lax.dot(
        x_f32.astype(jnp.bfloat16), p_bf16,
        preferred_element_type=jnp.float32) * INV_S2


def _mlp2(x, w1_ref, w2_ref):
    h = jnp.maximum(
        jnp.dot(x, w1_ref[...], preferred_element_type=jnp.float32), 0.0)
    return jnp.dot(h, w2_ref[...], preferred_element_type=jnp.float32)


def _l2n(x):
    n = jnp.sqrt(jnp.sum(x * x, axis=1, keepdims=True))
    return x / jnp.maximum(n, EPS)


def _ctx_body(x_ref, p_ref, kw1_ref, kw2_ref, vw1_ref, vw2_ref,
              kt_ref, v_ref):
    pooled = _pool(x_ref[...], p_ref[...])
    keys = _l2n(_mlp2(pooled, kw1_ref, kw2_ref))
    kt_ref[...] = keys.T
    v_ref[...] = _mlp2(pooled, vw1_ref, vw2_ref)


def _query_body(x_ref, p_ref, qw1_ref, qw2_ref, q_ref):
    pooled = _pool(x_ref[...], p_ref[...])
    q_ref[...] = _l2n(_mlp2(pooled, qw1_ref, qw2_ref)) * SOFTMAX_SCALE


def _attn_body(q_ref, kt_ref, v_ref, fw1_ref, fw2_ref, o_ref):
    s = jnp.dot(q_ref[...], kt_ref[...],
                preferred_element_type=jnp.float32)
    e = jnp.exp(s)
    denom = jnp.sum(e, axis=1, keepdims=True)
    attn = jnp.dot(e, v_ref[...],
                   preferred_element_type=jnp.float32) / denom
    o_ref[...] = _mlp2(attn, fw1_ref, fw2_ref)


def _full(shape):
    return pl.BlockSpec(shape, lambda i: tuple(0 for _ in shape))


def kernel(central_features, context_features, qw1, qw2, kw1, kw2,
           vw1, vw2, fw1, fw2, interpret=False):
    N = central_features.shape[0]
    T = context_features.shape[0]
    xc = central_features.reshape(N, F)
    xt = context_features.reshape(T, F)
    pmat = jnp.repeat(jnp.eye(C, dtype=jnp.float32), S2,
                      axis=0).astype(jnp.bfloat16)

    BT = 256
    kt, values = pl.pallas_call(
        _ctx_body,
        grid=(T // BT,),
        in_specs=[
            pl.BlockSpec((BT, F), lambda i: (i, 0)),
            _full((F, C)),
            _full(kw1.shape), _full(kw2.shape),
            _full(vw1.shape), _full(vw2.shape),
        ],
        out_specs=[
            pl.BlockSpec((QK, BT), lambda i: (0, i)),
            pl.BlockSpec((BT, VD), lambda i: (i, 0)),
        ],
        out_shape=[
            jax.ShapeDtypeStruct((QK, T), jnp.float32),
            jax.ShapeDtypeStruct((T, VD), jnp.float32),
        ],
        compiler_params=pltpu.CompilerParams(
            dimension_semantics=("parallel",),
            vmem_limit_bytes=52 * 1024 * 1024),
        interpret=interpret,
    )(xt, pmat, kw1, kw2, vw1, vw2)

    BQ = 256
    q = pl.pallas_call(
        _query_body,
        grid=(N // BQ,),
        in_specs=[
            pl.BlockSpec((BQ, F), lambda i: (i, 0)),
            _full((F, C)),
            _full(qw1.shape), _full(qw2.shape),
        ],
        out_specs=pl.BlockSpec((BQ, QK), lambda i: (i, 0)),
        out_shape=jax.ShapeDtypeStruct((N, QK), jnp.float32),
        compiler_params=pltpu.CompilerParams(
            dimension_semantics=("parallel",),
            vmem_limit_bytes=52 * 1024 * 1024),
        interpret=interpret,
    )(xc, pmat, qw1, qw2)

    BN = 128
    out = pl.pallas_call(
        _attn_body,
        grid=(N // BN,),
        in_specs=[
            pl.BlockSpec((BN, QK), lambda i: (i, 0)),
            _full((QK, T)),
            _full((T, VD)),
            _full(fw1.shape), _full(fw2.shape),
        ],
        out_specs=pl.BlockSpec((BN, C), lambda i: (i, 0)),
        out_shape=jax.ShapeDtypeStruct((N, C), jnp.float32),
        compiler_params=pltpu.CompilerParams(
            dimension_semantics=("parallel",),
            vmem_limit_bytes=52 * 1024 * 1024),
        interpret=interpret,
    )(q, kt, values, fw1, fw2)
    return out


# speedup vs baseline: 1.2690x; 1.2690x over previous
import jax
import jax.numpy as jnp
from jax.experimental import pallas as pl
from jax.experimental.pallas import tpu as pltpu

C = 256
QK = 256
VD = 256
S2 = 49
SOFTMAX_SCALE = 1.0 / (0.01 * C ** 0.5)
EPS = 1e-12
INV_S2 = 1.0 / S2
BR = 256


def _pool(x_ref, p_ref):
    return jax.lax.dot(
        p_ref[...], x_ref[...].astype(jnp.bfloat16),
        preferred_element_type=jnp.float32) * INV_S2


def _mlp2(x, w1_ref, w2_ref):
    h = jnp.maximum(
        jnp.dot(x, w1_ref[...], preferred_element_type=jnp.float32), 0.0)
    return jnp.dot(h, w2_ref[...], preferred_element_type=jnp.float32)


def _l2n(x):
    n = jnp.sqrt(jnp.sum(x * x, axis=1, keepdims=True))
    return x / jnp.maximum(n, EPS)


def _ctx_body(x_ref, p_ref, kw1_ref, kw2_ref, vw1_ref, vw2_ref,
              kt_ref, v_ref):
    pooled = _pool(x_ref, p_ref)
    keys = _l2n(_mlp2(pooled, kw1_ref, kw2_ref))
    kt_ref[...] = keys.T
    v_ref[...] = _mlp2(pooled, vw1_ref, vw2_ref)


def _query_body(x_ref, p_ref, qw1_ref, qw2_ref, q_ref):
    pooled = _pool(x_ref, p_ref)
    q_ref[...] = _l2n(_mlp2(pooled, qw1_ref, qw2_ref)) * SOFTMAX_SCALE


def _attn_body(q_ref, kt_ref, v_ref, fw1_ref, fw2_ref, o_ref):
    s = jnp.dot(q_ref[...], kt_ref[...],
                preferred_element_type=jnp.float32)
    e = jnp.exp(s)
    denom = jnp.sum(e, axis=1, keepdims=True)
    attn = jnp.dot(e, v_ref[...],
                   preferred_element_type=jnp.float32) / denom
    o_ref[...] = _mlp2(attn, fw1_ref, fw2_ref)


def _full(shape):
    return pl.BlockSpec(shape, lambda i: tuple(0 for _ in shape))


def kernel(central_features, context_features, qw1, qw2, kw1, kw2,
           vw1, vw2, fw1, fw2, interpret=False):
    N = central_features.shape[0]
    T = context_features.shape[0]
    xc = central_features.transpose(0, 2, 3, 1).reshape(N * S2, C)
    xt = context_features.transpose(0, 2, 3, 1).reshape(T * S2, C)
    pmat = jnp.repeat(jnp.eye(BR, dtype=jnp.float32), S2,
                      axis=1).astype(jnp.bfloat16)

    kt, values = pl.pallas_call(
        _ctx_body,
        grid=(T // BR,),
        in_specs=[
            pl.BlockSpec((BR * S2, C), lambda i: (i, 0)),
            _full((BR, BR * S2)),
            _full(kw1.shape), _full(kw2.shape),
            _full(vw1.shape), _full(vw2.shape),
        ],
        out_specs=[
            pl.BlockSpec((QK, BR), lambda i: (0, i)),
            pl.BlockSpec((BR, VD), lambda i: (i, 0)),
        ],
        out_shape=[
            jax.ShapeDtypeStruct((QK, T), jnp.float32),
            jax.ShapeDtypeStruct((T, VD), jnp.float32),
        ],
        compiler_params=pltpu.CompilerParams(
            dimension_semantics=("parallel",),
            vmem_limit_bytes=52 * 1024 * 1024),
        interpret=interpret,
    )(xt, pmat, kw1, kw2, vw1, vw2)

    q = pl.pallas_call(
        _query_body,
        grid=(N // BR,),
        in_specs=[
            pl.BlockSpec((BR * S2, C), lambda i: (i, 0)),
            _full((BR, BR * S2)),
            _full(qw1.shape), _full(qw2.shape),
        ],
        out_specs=pl.BlockSpec((BR, QK), lambda i: (i, 0)),
        out_shape=jax.ShapeDtypeStruct((N, QK), jnp.float32),
        compiler_params=pltpu.CompilerParams(
            dimension_semantics=("parallel",),
            vmem_limit_bytes=52 * 1024 * 1024),
        interpret=interpret,
    )(xc, pmat, qw1, qw2)

    BN = 128
    out = pl.pallas_call(
        _attn_body,
        grid=(N // BN,),
        in_specs=[
            pl.BlockSpec((BN, QK), lambda i: (i, 0)),
            _full((QK, T)),
            _full((T, VD)),
            _full(fw1.shape), _full(fw2.shape),
        ],
        out_specs=pl.BlockSpec((BN, C), lambda i: (i, 0)),
        out_shape=jax.ShapeDtypeStruct((N, C), jnp.float32),
        compiler_params=pltpu.CompilerParams(
            dimension_semantics=("parallel",),
            vmem_limit_bytes=52 * 1024 * 1024),
        interpret=interpret,
    )(q, kt, values, fw1, fw2)
    return out


# speedup vs baseline: 6.2999x; 4.9645x over previous
import jax
import jax.numpy as jnp
from jax.experimental import pallas as pl
from jax.experimental.pallas import tpu as pltpu

C = 256
QK = 256
VD = 256
S2 = 49
SOFTMAX_SCALE = 1.0 / (0.01 * C ** 0.5)
EPS = 1e-12
INV_S2 = 1.0 / S2


def _pool(x_ref):
    acc = x_ref[0]
    for s in range(1, S2):
        acc = acc + x_ref[s]
    return acc * INV_S2


def _mlp2(x, w1_ref, w2_ref):
    h = jnp.maximum(
        jnp.dot(x, w1_ref[...], preferred_element_type=jnp.float32), 0.0)
    return jnp.dot(h, w2_ref[...], preferred_element_type=jnp.float32)


def _l2n(x):
    n = jnp.sqrt(jnp.sum(x * x, axis=1, keepdims=True))
    return x / jnp.maximum(n, EPS)


def _ctx_body(x_ref, kw1_ref, kw2_ref, vw1_ref, vw2_ref, kt_ref, v_ref):
    pooled = _pool(x_ref)
    keys = _l2n(_mlp2(pooled, kw1_ref, kw2_ref))
    kt_ref[...] = keys.T
    v_ref[...] = _mlp2(pooled, vw1_ref, vw2_ref)


def _query_body(x_ref, qw1_ref, qw2_ref, q_ref):
    pooled = _pool(x_ref)
    q_ref[...] = _l2n(_mlp2(pooled, qw1_ref, qw2_ref)) * SOFTMAX_SCALE


def _attn_body(q_ref, kt_ref, v_ref, fw1_ref, fw2_ref, o_ref):
    s = jnp.dot(q_ref[...], kt_ref[...],
                preferred_element_type=jnp.float32)
    e = jnp.exp(s)
    denom = jnp.sum(e, axis=1, keepdims=True)
    attn = jnp.dot(e, v_ref[...],
                   preferred_element_type=jnp.float32) / denom
    o_ref[...] = _mlp2(attn, fw1_ref, fw2_ref)


def _full(shape):
    return pl.BlockSpec(shape, lambda i: tuple(0 for _ in shape))


def kernel(central_features, context_features, qw1, qw2, kw1, kw2,
           vw1, vw2, fw1, fw2, interpret=False):
    N = central_features.shape[0]
    T = context_features.shape[0]
    xc = central_features.transpose(2, 3, 0, 1).reshape(S2, N, C)
    xt = context_features.transpose(2, 3, 0, 1).reshape(S2, T, C)

    BT = 256
    kt, values = pl.pallas_call(
        _ctx_body,
        grid=(T // BT,),
        in_specs=[
            pl.BlockSpec((S2, BT, C), lambda i: (0, i, 0)),
            _full(kw1.shape), _full(kw2.shape),
            _full(vw1.shape), _full(vw2.shape),
        ],
        out_specs=[
            pl.BlockSpec((QK, BT), lambda i: (0, i)),
            pl.BlockSpec((BT, VD), lambda i: (i, 0)),
        ],
        out_shape=[
            jax.ShapeDtypeStruct((QK, T), jnp.float32),
            jax.ShapeDtypeStruct((T, VD), jnp.float32),
        ],
        compiler_params=pltpu.CompilerParams(
            dimension_semantics=("parallel",),
            vmem_limit_bytes=52 * 1024 * 1024),
        interpret=interpret,
    )(xt, kw1, kw2, vw1, vw2)

    BQ = 256
    q = pl.pallas_call(
        _query_body,
        grid=(N // BQ,),
        in_specs=[
            pl.BlockSpec((S2, BQ, C), lambda i: (0, i, 0)),
            _full(qw1.shape), _full(qw2.shape),
        ],
        out_specs=pl.BlockSpec((BQ, QK), lambda i: (i, 0)),
        out_shape=jax.ShapeDtypeStruct((N, QK), jnp.float32),
        compiler_params=pltpu.CompilerParams(
            dimension_semantics=("parallel",),
            vmem_limit_bytes=52 * 1024 * 1024),
        interpret=interpret,
    )(xc, qw1, qw2)

    BN = 128
    out = pl.pallas_call(
        _attn_body,
        grid=(N // BN,),
        in_specs=[
            pl.BlockSpec((BN, QK), lambda i: (i, 0)),
            _full((QK, T)),
            _full((T, VD)),
            _full(fw1.shape), _full(fw2.shape),
        ],
        out_specs=pl.BlockSpec((BN, C), lambda i: (i, 0)),
        out_shape=jax.ShapeDtypeStruct((N, C), jnp.float32),
        compiler_params=pltpu.CompilerParams(
            dimension_semantics=("parallel",),
            vmem_limit_bytes=52 * 1024 * 1024),
        interpret=interpret,
    )(q, kt, values, fw1, fw2)
    return out


# speedup vs baseline: 6.9124x; 1.0972x over previous
import jax
import jax.numpy as jnp
from jax.experimental import pallas as pl
from jax.experimental.pallas import tpu as pltpu

C = 256
QK = 256
VD = 256
S2 = 49
SOFTMAX_SCALE = 1.0 / (0.01 * C ** 0.5)
EPS = 1e-12
INV_S2 = 1.0 / S2


def _pool(x_ref):
    acc = x_ref[0]
    for s in range(1, S2):
        acc = acc + x_ref[s]
    return acc * INV_S2


def _mlp2(x, w1_ref, w2_ref):
    h = jnp.maximum(
        jnp.dot(x, w1_ref[...], preferred_element_type=jnp.float32), 0.0)
    return jnp.dot(h, w2_ref[...], preferred_element_type=jnp.float32)


def _l2n(x):
    n = jnp.sqrt(jnp.sum(x * x, axis=1, keepdims=True))
    return x / jnp.maximum(n, EPS)


def _ctx_body(x_ref, kw1_ref, kw2_ref, vw1_ref, vw2_ref, kt_ref, v_ref):
    pooled = _pool(x_ref)
    keys = _l2n(_mlp2(pooled, kw1_ref, kw2_ref))
    kt_ref[...] = keys.T
    v_ref[...] = _mlp2(pooled, vw1_ref, vw2_ref)


def _qattn_body(x_ref, qw1_ref, qw2_ref, kt_ref, v_ref, fw1_ref, fw2_ref,
                o_ref):
    pooled = _pool(x_ref)
    q = _l2n(_mlp2(pooled, qw1_ref, qw2_ref)) * SOFTMAX_SCALE
    s = jnp.dot(q, kt_ref[...],
                preferred_element_type=jnp.float32)
    e = jnp.exp(s)
    denom = jnp.sum(e, axis=1, keepdims=True)
    attn = jnp.dot(e, v_ref[...],
                   preferred_element_type=jnp.float32) / denom
    o_ref[...] = _mlp2(attn, fw1_ref, fw2_ref)


def _full(shape):
    return pl.BlockSpec(shape, lambda i: tuple(0 for _ in shape))


def kernel(central_features, context_features, qw1, qw2, kw1, kw2,
           vw1, vw2, fw1, fw2, interpret=False):
    N = central_features.shape[0]
    T = context_features.shape[0]
    xc = central_features.transpose(2, 3, 0, 1).reshape(S2, N, C)
    xt = context_features.transpose(2, 3, 0, 1).reshape(S2, T, C)

    BT = 256
    kt, values = pl.pallas_call(
        _ctx_body,
        grid=(T // BT,),
        in_specs=[
            pl.BlockSpec((S2, BT, C), lambda i: (0, i, 0)),
            _full(kw1.shape), _full(kw2.shape),
            _full(vw1.shape), _full(vw2.shape),
        ],
        out_specs=[
            pl.BlockSpec((QK, BT), lambda i: (0, i)),
            pl.BlockSpec((BT, VD), lambda i: (i, 0)),
        ],
        out_shape=[
            jax.ShapeDtypeStruct((QK, T), jnp.float32),
            jax.ShapeDtypeStruct((T, VD), jnp.float32),
        ],
        compiler_params=pltpu.CompilerParams(
            dimension_semantics=("parallel",),
            vmem_limit_bytes=52 * 1024 * 1024),
        interpret=interpret,
    )(xt, kw1, kw2, vw1, vw2)

    BQ = 128
    out = pl.pallas_call(
        _qattn_body,
        grid=(N // BQ,),
        in_specs=[
            pl.BlockSpec((S2, BQ, C), lambda i: (0, i, 0)),
            _full(qw1.shape), _full(qw2.shape),
            _full((QK, T)),
            _full((T, VD)),
            _full(fw1.shape), _full(fw2.shape),
        ],
        out_specs=pl.BlockSpec((BQ, C), lambda i: (i, 0)),
        out_shape=jax.ShapeDtypeStruct((N, C), jnp.float32),
        compiler_params=pltpu.CompilerParams(
            dimension_semantics=("parallel",),
            vmem_limit_bytes=56 * 1024 * 1024),
        interpret=interpret,
    )(xc, qw1, qw2, kt, values, fw1, fw2)
    return out
